# baseline (device time: 28837 ns/iter reference)
import jax
import jax.numpy as jnp
from jax import lax
from jax.experimental import pallas as pl
from jax.experimental.pallas import tpu as pltpu

K = 16
NEG = float("-inf")


def kernel(x):
    m, n_loc = x.shape

    def body(x_ref, out_ref, mine_ref, peer_ref, send_sem, recv_sem):
        my_x = lax.axis_index("x")
        my_y = lax.axis_index("y")
        peer = (1 - my_x, my_y)

        barrier_sem = pltpu.get_barrier_semaphore()
        pl.semaphore_signal(
            barrier_sem, inc=1, device_id=peer,
            device_id_type=pl.DeviceIdType.MESH,
        )
        pl.semaphore_wait(barrier_sem, 1)

        C = 7
        W = 128
        G = n_loc // W
        vals2 = x_ref[...].reshape(m * G, W)
        gmax = jnp.max(vals2, axis=1, keepdims=True)
        cands = []
        for c in range(C):
            cands.append(gmax.reshape(m, G))
            if c < C - 1:
                vals2 = jnp.where(vals2 == gmax, NEG, vals2)
                gmax = jnp.max(vals2, axis=1, keepdims=True)
        cand = jnp.concatenate(cands, axis=1)

        for k in range(K):
            mx = jnp.max(cand, axis=1, keepdims=True)
            mine_ref[:, k : k + 1] = mx
            if k < K - 1:
                cand = jnp.where(cand == mx, NEG, cand)

        rdma = pltpu.make_async_remote_copy(
            src_ref=mine_ref,
            dst_ref=peer_ref,
            send_sem=send_sem,
            recv_sem=recv_sem,
            device_id=peer,
            device_id_type=pl.DeviceIdType.MESH,
        )
        rdma.start()
        rdma.wait()

        cand = jnp.concatenate([mine_ref[...], peer_ref[...]], axis=1)
        for k in range(K):
            mx = jnp.max(cand, axis=1, keepdims=True)
            out_ref[:, k : k + 1] = mx
            if k < K - 1:
                cand = jnp.where(cand == mx, NEG, cand)

    return pl.pallas_call(
        body,
        out_shape=jax.ShapeDtypeStruct((m, K), jnp.float32),
        in_specs=[pl.BlockSpec(memory_space=pltpu.VMEM)],
        out_specs=pl.BlockSpec(memory_space=pltpu.VMEM),
        scratch_shapes=[
            pltpu.VMEM((m, K), jnp.float32),
            pltpu.VMEM((m, K), jnp.float32),
            pltpu.SemaphoreType.DMA,
            pltpu.SemaphoreType.DMA,
        ],
        compiler_params=pltpu.CompilerParams(collective_id=0),
    )(x)
